# baseline (speedup 1.0000x reference)
import jax
import jax.numpy as jnp
from jax import lax
from jax.experimental import pallas as pl
from jax.experimental.pallas import tpu as pltpu
from jax.experimental.pallas import tpu_sc as plsc

N = 10000
E = 320000
D = 128
B = 512
STEPS = 3

NC = 2
NS = 16
NW = NC * NS
C = 80
CE = 128
EPAD = -(-E // (NW * CE)) * NW * CE
ECE = EPAD // NW // CE
PAD_N = ((N + NW * C - 1) // (NW * C)) * (NW * C)
GC = PAD_N // NW // C
ZROWS = 1000
DROW = N


def _sc_mesh():
    return plsc.VectorSubcoreMesh(core_axis_name="c", subcore_axis_name="s",
                                  num_cores=NC, num_subcores=NS)


_PREC = jax.lax.Precision.HIGHEST


def _sc_gather_deg_body(emb_h, xpad_h, dst_h, zeros8_h, ones8_h,
                        h0_h, degp_h, idx_v0, idx_v1, idxe_v0, idxe_v1,
                        rows_v0, rows_v1, ones_v, deg_s, sem0, sem1):
    c = lax.axis_index("c")
    s = lax.axis_index("s")
    wid = s * NC + c
    idx_v = (idx_v0, idx_v1)
    idxe_v = (idxe_v0, idxe_v1)
    rows_v = (rows_v0, rows_v1)
    sems = (sem0, sem1)
    pltpu.sync_copy(ones8_h, ones_v)

    @pl.when(s < N // ZROWS)
    def _():
        pltpu.sync_copy(zeros8_h, deg_s.at[pl.ds(s * ZROWS, ZROWS)])
    plsc.subcore_barrier()

    def ggroup(g, carry):
        cps = []
        for b in range(2):
            base = pl.multiple_of(wid * (GC * C) + (g * 2 + b) * C, 8)
            pltpu.sync_copy(xpad_h.at[pl.ds(base, C)], idx_v[b])
            cps.append(pltpu.async_copy(emb_h.at[idx_v[b]], rows_v[b],
                                        sems[b]))
        for b in range(2):
            base = pl.multiple_of(wid * (GC * C) + (g * 2 + b) * C, 8)
            cps[b].wait()
            pltpu.sync_copy(rows_v[b], h0_h.at[pl.ds(base, C)])
        return carry
    lax.fori_loop(0, GC // 2, ggroup, 0)

    def dgroup(g, carry):
        cps = []
        for b in range(2):
            base = pl.multiple_of(wid * (ECE * CE) + (g * 2 + b) * CE, 8)
            pltpu.sync_copy(dst_h.at[pl.ds(base, CE)], idxe_v[b])
            cps.append(pltpu.async_copy(ones_v, deg_s.at[idxe_v[b]],
                                        sems[b], add=True))
        for b in range(2):
            cps[b].wait()
        return carry
    lax.fori_loop(0, ECE // 2, dgroup, 0)
    plsc.subcore_barrier()

    @pl.when(s < N // ZROWS)
    def _():
        pltpu.sync_copy(deg_s.at[pl.ds(s * ZROWS, ZROWS)],
                        degp_h.at[c, pl.ds(s * ZROWS, ZROWS)])


def _sc_gather_deg(emb, xpad, dst, zeros8, ones8):
    return pl.kernel(
        _sc_gather_deg_body,
        out_type=[
            jax.ShapeDtypeStruct((PAD_N, D), jnp.float32),
            jax.ShapeDtypeStruct((NC, N, 8), jnp.float32),
        ],
        mesh=_sc_mesh(),
        scratch_types=[
            pltpu.VMEM((C,), jnp.int32),
            pltpu.VMEM((C,), jnp.int32),
            pltpu.VMEM((CE,), jnp.int32),
            pltpu.VMEM((CE,), jnp.int32),
            pltpu.VMEM((C, D), jnp.float32),
            pltpu.VMEM((C, D), jnp.float32),
            pltpu.VMEM((CE, 8), jnp.float32),
            pltpu.VMEM_SHARED((N + 8, 8), jnp.float32),
            pltpu.SemaphoreType.DMA,
            pltpu.SemaphoreType.DMA,
        ],
    )(emb, xpad, dst, zeros8, ones8)


def _sc_edge_agg_body(h_h, src_h, dst_h, zerosd_h, aggp_h,
                      idx_s0, idx_s1, idx_s2, idx_d0, idx_d1, idx_d2,
                      rows_v0, rows_v1, rows_v2, agg_s, sem0, sem1, sem2):
    c = lax.axis_index("c")
    s = lax.axis_index("s")
    wid = s * NC + c
    idx_s = (idx_s0, idx_s1, idx_s2)
    idx_d = (idx_d0, idx_d1, idx_d2)
    rows_v = (rows_v0, rows_v1, rows_v2)
    sems = (sem0, sem1, sem2)

    @pl.when(s < N // ZROWS)
    def _():
        pltpu.sync_copy(zerosd_h, agg_s.at[pl.ds(s * ZROWS, ZROWS)])
    plsc.subcore_barrier()

    def egroup(g, carry):
        j0 = g * 3
        cps = []
        for b in range(3):
            base = pl.multiple_of(wid * (ECE * CE) + (j0 + b) * CE, 8)
            pltpu.sync_copy(src_h.at[pl.ds(base, CE)], idx_s[b])
            cps.append(pltpu.async_copy(h_h.at[idx_s[b]], rows_v[b],
                                        sems[b]))
        for b in range(3):
            base = pl.multiple_of(wid * (ECE * CE) + (j0 + b) * CE, 8)
            pltpu.sync_copy(dst_h.at[pl.ds(base, CE)], idx_d[b])
        for b in range(3):
            cps[b].wait()
            pltpu.sync_copy(rows_v[b], agg_s.at[idx_d[b]], add=True)
        return carry
    lax.fori_loop(0, ECE // 3, egroup, 0)
    for j in range(ECE - ECE % 3, ECE):
        base = pl.multiple_of(wid * (ECE * CE) + j * CE, 8)
        pltpu.sync_copy(src_h.at[pl.ds(base, CE)], idx_s0)
        cp = pltpu.async_copy(h_h.at[idx_s0], rows_v0, sem0)
        pltpu.sync_copy(dst_h.at[pl.ds(base, CE)], idx_d0)
        cp.wait()
        pltpu.sync_copy(rows_v0, agg_s.at[idx_d0], add=True)
    plsc.subcore_barrier()

    @pl.when(s < N // ZROWS)
    def _():
        pltpu.sync_copy(agg_s.at[pl.ds(s * ZROWS, ZROWS)],
                        aggp_h.at[c, pl.ds(s * ZROWS, ZROWS)])


def _sc_edge_agg(h, src, dst, zerosd):
    return pl.kernel(
        _sc_edge_agg_body,
        out_type=jax.ShapeDtypeStruct((NC, N, D), jnp.float32),
        mesh=_sc_mesh(),
        scratch_types=[
            pltpu.VMEM((CE,), jnp.int32),
            pltpu.VMEM((CE,), jnp.int32),
            pltpu.VMEM((CE,), jnp.int32),
            pltpu.VMEM((CE,), jnp.int32),
            pltpu.VMEM((CE,), jnp.int32),
            pltpu.VMEM((CE,), jnp.int32),
            pltpu.VMEM((CE, D), jnp.float32),
            pltpu.VMEM((CE, D), jnp.float32),
            pltpu.VMEM((CE, D), jnp.float32),
            pltpu.VMEM_SHARED((N + 8, D), jnp.float32),
            pltpu.SemaphoreType.DMA,
            pltpu.SemaphoreType.DMA,
            pltpu.SemaphoreType.DMA,
        ],
    )(h, src, dst, zerosd)


_BN = 400
_GRID = N // _BN


def _sage_body(aggp_ref, degp_ref, h_ref, wl_ref, wr_ref, bl_ref, out_ref):
    agg = aggp_ref[0] + aggp_ref[1]
    deg = jnp.maximum(degp_ref[0, :, 0:1] + degp_ref[1, :, 0:1], 1.0)
    out = (jnp.dot(agg / deg, wl_ref[...], precision=_PREC,
                   preferred_element_type=jnp.float32)
           + bl_ref[...]
           + jnp.dot(h_ref[...], wr_ref[...], precision=_PREC,
                     preferred_element_type=jnp.float32))
    nrm = jnp.maximum(jnp.sqrt(jnp.sum(out * out, axis=1, keepdims=True)),
                      1e-12)
    out_ref[...] = jnp.maximum(out / nrm, 0.0)


def _tc_sage(aggp, degp, h, wl, wr, bl):
    return pl.pallas_call(
        _sage_body,
        grid=(_GRID,),
        in_specs=[
            pl.BlockSpec((NC, _BN, D), lambda i: (0, i, 0)),
            pl.BlockSpec((NC, _BN, 8), lambda i: (0, i, 0)),
            pl.BlockSpec((_BN, D), lambda i: (i, 0)),
            pl.BlockSpec((D, D), lambda i: (0, 0)),
            pl.BlockSpec((D, D), lambda i: (0, 0)),
            pl.BlockSpec((1, D), lambda i: (0, 0)),
        ],
        out_specs=pl.BlockSpec((_BN, D), lambda i: (i, 0)),
        out_shape=jax.ShapeDtypeStruct((N, D), jnp.float32),
    )(aggp, degp, h, wl, wr, bl)


def _lstm_math(hh, cc, r_num, r_den, wih, whh, bih, bhh):
    r = r_num / jnp.where(r_den > 0.0, r_den, 1.0)
    q_star = jnp.concatenate([hh, r], axis=1)
    gates = (lax.dot_general(q_star, wih, (((1,), (1,)), ((), ())),
                             precision=_PREC,
                             preferred_element_type=jnp.float32)
             + bih
             + lax.dot_general(hh, whh, (((1,), (1,)), ((), ())),
                               precision=_PREC,
                               preferred_element_type=jnp.float32)
             + bhh)
    ig = jax.nn.sigmoid(gates[:, 0 * D:1 * D])
    fg = jax.nn.sigmoid(gates[:, 1 * D:2 * D])
    gg = jnp.tanh(gates[:, 2 * D:3 * D])
    og = jax.nn.sigmoid(gates[:, 3 * D:4 * D])
    cc2 = fg * cc + ig * gg
    return og * jnp.tanh(cc2), cc2


def _make_attn_body(with_mlp):
    def body(h_ref, b_ref, hh_ref, cc_ref, rp_ref, denp_ref,
             wih_ref, whh_ref, bih_ref, bhh_ref,
             w1_ref, b1_ref, w2_ref, b2_ref, w3_ref, b3_ref,
             r_ref, den_ref, hho_ref, cco_ref, out_ref, q_scr):
        i = pl.program_id(0)

        @pl.when(i == 0)
        def _():
            hh2, cc2 = _lstm_math(hh_ref[...], cc_ref[...], rp_ref[...],
                                  denp_ref[...], wih_ref[...], whh_ref[...],
                                  bih_ref[...], bhh_ref[...])
            hho_ref[...] = hh2
            cco_ref[...] = cc2
            q_scr[...] = hh2
            r_ref[...] = jnp.zeros_like(r_ref)
            den_ref[...] = jnp.zeros_like(den_ref)
        onehot = (b_ref[...] == lax.broadcasted_iota(jnp.int32, (_BN, B), 1)
                  ).astype(jnp.float32)
        h = h_ref[...]
        qb = jnp.dot(onehot, q_scr[...], precision=_PREC,
                     preferred_element_type=jnp.float32)
        e = jnp.sum(h * qb, axis=1, keepdims=True)
        a = jnp.exp(e)
        den_ref[...] += lax.dot_general(onehot, a, (((0,), (0,)), ((), ())),
                                        precision=_PREC,
                                        preferred_element_type=jnp.float32)
        r_ref[...] += lax.dot_general(onehot, a * h,
                                      (((0,), (0,)), ((), ())),
                                      precision=_PREC,
                                      preferred_element_type=jnp.float32)
        if with_mlp:
            @pl.when(i == _GRID - 1)
            def _():
                den = den_ref[...]
                r = r_ref[...] / jnp.where(den > 0.0, den, 1.0)
                q_star = jnp.concatenate([q_scr[...], r], axis=1)
                z = jnp.maximum(
                    jnp.dot(q_star, w1_ref[...], precision=_PREC,
                            preferred_element_type=jnp.float32)
                    + b1_ref[...], 0.0)
                z = jnp.maximum(
                    jnp.dot(z, w2_ref[...], precision=_PREC,
                            preferred_element_type=jnp.float32)
                    + b2_ref[...], 0.0)
                z = jnp.dot(z, w3_ref[...], precision=_PREC,
                            preferred_element_type=jnp.float32) + b3_ref[...]
                out_ref[...] = jax.nn.sigmoid(z)
        else:
            out_ref[...] = jnp.zeros_like(out_ref)
    return body


def _tc_attn_step(h, batch2, hh, cc, rp, denp, wih, whh, bih, bhh,
                  w1, b1, w2, b2, w3, b3, with_mlp):
    full = lambda i: (0, 0)
    return pl.pallas_call(
        _make_attn_body(with_mlp),
        grid=(_GRID,),
        in_specs=[
            pl.BlockSpec((_BN, D), lambda i: (i, 0)),
            pl.BlockSpec((_BN, 1), lambda i: (i, 0)),
            pl.BlockSpec((B, D), full),
            pl.BlockSpec((B, D), full),
            pl.BlockSpec((B, D), full),
            pl.BlockSpec((B, 1), full),
            pl.BlockSpec((4 * D, 2 * D), full),
            pl.BlockSpec((4 * D, D), full),
            pl.BlockSpec((1, 4 * D), full),
            pl.BlockSpec((1, 4 * D), full),
            pl.BlockSpec((2 * D, D), full),
            pl.BlockSpec((1, D), full),
            pl.BlockSpec((D, D // 2), full),
            pl.BlockSpec((1, D // 2), full),
            pl.BlockSpec((D // 2, 1), full),
            pl.BlockSpec((1, 1), full),
        ],
        out_specs=[
            pl.BlockSpec((B, D), full),
            pl.BlockSpec((B, 1), full),
            pl.BlockSpec((B, D), full),
            pl.BlockSpec((B, D), full),
            pl.BlockSpec((B, 1), full),
        ],
        out_shape=[
            jax.ShapeDtypeStruct((B, D), jnp.float32),
            jax.ShapeDtypeStruct((B, 1), jnp.float32),
            jax.ShapeDtypeStruct((B, D), jnp.float32),
            jax.ShapeDtypeStruct((B, D), jnp.float32),
            jax.ShapeDtypeStruct((B, 1), jnp.float32),
        ],
        scratch_shapes=[pltpu.VMEM((B, D), jnp.float32)],
    )(h, batch2, hh, cc, rp, denp, wih, whh, bih, bhh,
      w1, b1, w2, b2, w3, b3)


def _lstm_body(hh_ref, cc_ref, r_ref, den_ref, wih_ref, whh_ref,
               bih_ref, bhh_ref, hho_ref, cco_ref):
    hh = hh_ref[...]
    den = den_ref[...]
    r = r_ref[...] / jnp.where(den > 0.0, den, 1.0)
    q_star = jnp.concatenate([hh, r], axis=1)
    gates = (lax.dot_general(q_star, wih_ref[...], (((1,), (1,)), ((), ())),
                             precision=_PREC,
                             preferred_element_type=jnp.float32)
             + bih_ref[...]
             + lax.dot_general(hh, whh_ref[...], (((1,), (1,)), ((), ())),
                               precision=_PREC,
                               preferred_element_type=jnp.float32)
             + bhh_ref[...])
    ig = jax.nn.sigmoid(gates[:, 0 * D:1 * D])
    fg = jax.nn.sigmoid(gates[:, 1 * D:2 * D])
    gg = jnp.tanh(gates[:, 2 * D:3 * D])
    og = jax.nn.sigmoid(gates[:, 3 * D:4 * D])
    cc = fg * cc_ref[...] + ig * gg
    cco_ref[...] = cc
    hho_ref[...] = og * jnp.tanh(cc)


def _tc_lstm(hh, cc, r, den, wih, whh, bih, bhh):
    return pl.pallas_call(
        _lstm_body,
        out_shape=[
            jax.ShapeDtypeStruct((B, D), jnp.float32),
            jax.ShapeDtypeStruct((B, D), jnp.float32),
        ],
    )(hh, cc, r, den, wih, whh, bih, bhh)


def _mlp_body(hh_ref, r_ref, den_ref, w1_ref, b1_ref, w2_ref, b2_ref,
              w3_ref, b3_ref, out_ref):
    den = den_ref[...]
    r = r_ref[...] / jnp.where(den > 0.0, den, 1.0)
    q_star = jnp.concatenate([hh_ref[...], r], axis=1)
    z = jnp.maximum(jnp.dot(q_star, w1_ref[...], precision=_PREC,
                            preferred_element_type=jnp.float32)
                    + b1_ref[...], 0.0)
    z = jnp.maximum(jnp.dot(z, w2_ref[...], precision=_PREC,
                            preferred_element_type=jnp.float32)
                    + b2_ref[...], 0.0)
    z = jnp.dot(z, w3_ref[...], precision=_PREC,
                preferred_element_type=jnp.float32) + b3_ref[...]
    out_ref[...] = jax.nn.sigmoid(z)


def _tc_mlp(hh, r, den, w1, b1, w2, b2, w3, b3):
    return pl.pallas_call(
        _mlp_body,
        out_shape=jax.ShapeDtypeStruct((B, 1), jnp.float32),
    )(hh, r, den, w1, b1, w2, b2, w3, b3)


def kernel(x, edge_index, batch, emb, W_l0, b_l0, W_r0, W_l1, b_l1, W_r1,
           W_l2, b_l2, W_r2, W_ih, W_hh, b_ih, b_hh, Wc1, bc1, Wc2, bc2,
           Wc3, bc3):
    xpad = jnp.concatenate(
        [x[:, 0], jnp.zeros((PAD_N - N,), jnp.int32)])
    src = jnp.concatenate(
        [edge_index[0], jnp.zeros((EPAD - E,), jnp.int32)])
    dst = jnp.concatenate(
        [edge_index[1], jnp.full((EPAD - E,), DROW, jnp.int32)])
    batch2 = batch.reshape(N, 1)
    zeros8 = jnp.zeros((ZROWS, 8), jnp.float32)
    ones8 = jnp.ones((CE, 8), jnp.float32)
    zerosd = jnp.zeros((ZROWS, D), jnp.float32)

    h, degp = _sc_gather_deg(emb, xpad, dst, zeros8, ones8)

    for (Wl, bl, Wr) in ((W_l0, b_l0, W_r0), (W_l1, b_l1, W_r1),
                         (W_l2, b_l2, W_r2)):
        aggp = _sc_edge_agg(h, src, dst, zerosd)
        h = _tc_sage(aggp, degp, h, Wl, Wr, bl.reshape(1, D))

    hh = jnp.zeros((B, D), jnp.float32)
    cc = jnp.zeros((B, D), jnp.float32)
    r = jnp.zeros((B, D), jnp.float32)
    den = jnp.ones((B, 1), jnp.float32)
    bih2 = b_ih.reshape(1, 4 * D)
    bhh2 = b_hh.reshape(1, 4 * D)
    bc12 = bc1.reshape(1, D)
    bc22 = bc2.reshape(1, D // 2)
    bc32 = bc3.reshape(1, 1)
    out = None
    for step in range(STEPS):
        r, den, hh, cc, out = _tc_attn_step(
            h, batch2, hh, cc, r, den, W_ih, W_hh, bih2, bhh2,
            Wc1, bc12, Wc2, bc22, Wc3, bc32, step == STEPS - 1)
    return out[:, 0]


# speedup vs baseline: 1.3732x; 1.3732x over previous
import jax
import jax.numpy as jnp
from jax import lax
from jax.experimental import pallas as pl
from jax.experimental.pallas import tpu as pltpu
from jax.experimental.pallas import tpu_sc as plsc

N = 10000
E = 320000
D = 128
B = 512
STEPS = 3

NC = 2
NS = 16
NW = NC * NS
C = 80
CE = C
EPAD = E
ECE = EPAD // NW // CE
PAD_N = ((N + NW * C - 1) // (NW * C)) * (NW * C)
GC = PAD_N // NW // C
ZROWS = 1000


def _sc_mesh():
    return plsc.VectorSubcoreMesh(core_axis_name="c", subcore_axis_name="s",
                                  num_cores=NC, num_subcores=NS)


_PREC = jax.lax.Precision.HIGHEST


def _sc_gather_deg_body(emb_h, xpad_h, dst_h, zeros8_h, ones8_h,
                        h0_h, degp_h, idx_v0, idx_v1, idxe_v0, idxe_v1,
                        rows_v0, rows_v1, ones_v, deg_s, sem0, sem1):
    c = lax.axis_index("c")
    s = lax.axis_index("s")
    wid = s * NC + c
    idx_v = (idx_v0, idx_v1)
    idxe_v = (idxe_v0, idxe_v1)
    rows_v = (rows_v0, rows_v1)
    sems = (sem0, sem1)
    pltpu.sync_copy(ones8_h, ones_v)

    @pl.when(s < N // ZROWS)
    def _():
        pltpu.sync_copy(zeros8_h, deg_s.at[pl.ds(s * ZROWS, ZROWS)])
    plsc.subcore_barrier()

    def ggroup(g, carry):
        cps = []
        for b in range(2):
            base = pl.multiple_of(wid * (GC * C) + (g * 2 + b) * C, 8)
            pltpu.sync_copy(xpad_h.at[pl.ds(base, C)], idx_v[b])
            cps.append(pltpu.async_copy(emb_h.at[idx_v[b]], rows_v[b],
                                        sems[b]))
        for b in range(2):
            base = pl.multiple_of(wid * (GC * C) + (g * 2 + b) * C, 8)
            cps[b].wait()
            pltpu.sync_copy(rows_v[b], h0_h.at[pl.ds(base, C)])
        return carry
    lax.fori_loop(0, GC // 2, ggroup, 0)

    def dgroup(g, carry):
        cps = []
        for b in range(2):
            base = pl.multiple_of(wid * (ECE * CE) + (g * 2 + b) * CE, 8)
            pltpu.sync_copy(dst_h.at[pl.ds(base, CE)], idxe_v[b])
            cps.append(pltpu.async_copy(ones_v, deg_s.at[idxe_v[b]],
                                        sems[b], add=True))
        for b in range(2):
            cps[b].wait()
        return carry
    lax.fori_loop(0, ECE // 2, dgroup, 0)
    for j in range(ECE - ECE % 2, ECE):
        base = pl.multiple_of(wid * (ECE * CE) + j * CE, 8)
        pltpu.sync_copy(dst_h.at[pl.ds(base, CE)], idxe_v0)
        pltpu.sync_copy(ones_v, deg_s.at[idxe_v0], add=True)
    plsc.subcore_barrier()

    @pl.when(s < N // ZROWS)
    def _():
        pltpu.sync_copy(deg_s.at[pl.ds(s * ZROWS, ZROWS)],
                        degp_h.at[c, pl.ds(s * ZROWS, ZROWS)])


def _sc_gather_deg(emb, xpad, dst, zeros8, ones8):
    return pl.kernel(
        _sc_gather_deg_body,
        out_type=[
            jax.ShapeDtypeStruct((PAD_N, D), jnp.float32),
            jax.ShapeDtypeStruct((NC, N, 8), jnp.float32),
        ],
        mesh=_sc_mesh(),
        scratch_types=[
            pltpu.VMEM((C,), jnp.int32),
            pltpu.VMEM((C,), jnp.int32),
            pltpu.VMEM((CE,), jnp.int32),
            pltpu.VMEM((CE,), jnp.int32),
            pltpu.VMEM((C, D), jnp.float32),
            pltpu.VMEM((C, D), jnp.float32),
            pltpu.VMEM((CE, 8), jnp.float32),
            pltpu.VMEM_SHARED((N, 8), jnp.float32),
            pltpu.SemaphoreType.DMA,
            pltpu.SemaphoreType.DMA,
        ],
    )(emb, xpad, dst, zeros8, ones8)


def _sc_edge_agg_body(h_h, src_h, dst_h, zerosd_h, aggp_h,
                      idx_s0, idx_s1, idx_s2, idx_d0, idx_d1, idx_d2,
                      rows_v0, rows_v1, rows_v2, agg_s, sem0, sem1, sem2):
    c = lax.axis_index("c")
    s = lax.axis_index("s")
    wid = s * NC + c
    idx_s = (idx_s0, idx_s1, idx_s2)
    idx_d = (idx_d0, idx_d1, idx_d2)
    rows_v = (rows_v0, rows_v1, rows_v2)
    sems = (sem0, sem1, sem2)

    @pl.when(s < N // ZROWS)
    def _():
        pltpu.sync_copy(zerosd_h, agg_s.at[pl.ds(s * ZROWS, ZROWS)])
    plsc.subcore_barrier()

    def egroup(g, carry):
        j0 = g * 3
        cps = []
        for b in range(3):
            base = pl.multiple_of(wid * (ECE * CE) + (j0 + b) * CE, 8)
            pltpu.sync_copy(src_h.at[pl.ds(base, CE)], idx_s[b])
            cps.append(pltpu.async_copy(h_h.at[idx_s[b]], rows_v[b],
                                        sems[b]))
        for b in range(3):
            base = pl.multiple_of(wid * (ECE * CE) + (j0 + b) * CE, 8)
            pltpu.sync_copy(dst_h.at[pl.ds(base, CE)], idx_d[b])
        for b in range(3):
            cps[b].wait()
            pltpu.sync_copy(rows_v[b], agg_s.at[idx_d[b]], add=True)
        return carry
    lax.fori_loop(0, ECE // 3, egroup, 0)
    for j in range(ECE - ECE % 3, ECE):
        base = pl.multiple_of(wid * (ECE * CE) + j * CE, 8)
        pltpu.sync_copy(src_h.at[pl.ds(base, CE)], idx_s0)
        cp = pltpu.async_copy(h_h.at[idx_s0], rows_v0, sem0)
        pltpu.sync_copy(dst_h.at[pl.ds(base, CE)], idx_d0)
        cp.wait()
        pltpu.sync_copy(rows_v0, agg_s.at[idx_d0], add=True)
    plsc.subcore_barrier()

    @pl.when(s < N // ZROWS)
    def _():
        pltpu.sync_copy(agg_s.at[pl.ds(s * ZROWS, ZROWS)],
                        aggp_h.at[c, pl.ds(s * ZROWS, ZROWS)])


def _sc_edge_agg(h, src, dst, zerosd):
    return pl.kernel(
        _sc_edge_agg_body,
        out_type=jax.ShapeDtypeStruct((NC, N, D), jnp.float32),
        mesh=_sc_mesh(),
        scratch_types=[
            pltpu.VMEM((CE,), jnp.int32),
            pltpu.VMEM((CE,), jnp.int32),
            pltpu.VMEM((CE,), jnp.int32),
            pltpu.VMEM((CE,), jnp.int32),
            pltpu.VMEM((CE,), jnp.int32),
            pltpu.VMEM((CE,), jnp.int32),
            pltpu.VMEM((CE, D), jnp.float32),
            pltpu.VMEM((CE, D), jnp.float32),
            pltpu.VMEM((CE, D), jnp.float32),
            pltpu.VMEM_SHARED((N, D), jnp.float32),
            pltpu.SemaphoreType.DMA,
            pltpu.SemaphoreType.DMA,
            pltpu.SemaphoreType.DMA,
        ],
    )(h, src, dst, zerosd)


_BN = 400
_GRID = N // _BN


def _sage_body(aggp_ref, degp_ref, h_ref, wl_ref, wr_ref, bl_ref, out_ref):
    agg = aggp_ref[0] + aggp_ref[1]
    deg = jnp.maximum(degp_ref[0, :, 0:1] + degp_ref[1, :, 0:1], 1.0)
    out = (jnp.dot(agg / deg, wl_ref[...], precision=_PREC,
                   preferred_element_type=jnp.float32)
           + bl_ref[...]
           + jnp.dot(h_ref[...], wr_ref[...], precision=_PREC,
                     preferred_element_type=jnp.float32))
    nrm = jnp.maximum(jnp.sqrt(jnp.sum(out * out, axis=1, keepdims=True)),
                      1e-12)
    out_ref[...] = jnp.maximum(out / nrm, 0.0)


def _tc_sage(aggp, degp, h, wl, wr, bl):
    return pl.pallas_call(
        _sage_body,
        grid=(_GRID,),
        in_specs=[
            pl.BlockSpec((NC, _BN, D), lambda i: (0, i, 0)),
            pl.BlockSpec((NC, _BN, 8), lambda i: (0, i, 0)),
            pl.BlockSpec((_BN, D), lambda i: (i, 0)),
            pl.BlockSpec((D, D), lambda i: (0, 0)),
            pl.BlockSpec((D, D), lambda i: (0, 0)),
            pl.BlockSpec((1, D), lambda i: (0, 0)),
        ],
        out_specs=pl.BlockSpec((_BN, D), lambda i: (i, 0)),
        out_shape=jax.ShapeDtypeStruct((N, D), jnp.float32),
    )(aggp, degp, h, wl, wr, bl)


def _lstm_math(hh, cc, r_num, r_den, wih, whh, bih, bhh):
    r = r_num / jnp.where(r_den > 0.0, r_den, 1.0)
    q_star = jnp.concatenate([hh, r], axis=1)
    gates = (lax.dot_general(q_star, wih, (((1,), (1,)), ((), ())),
                             precision=_PREC,
                             preferred_element_type=jnp.float32)
             + bih
             + lax.dot_general(hh, whh, (((1,), (1,)), ((), ())),
                               precision=_PREC,
                               preferred_element_type=jnp.float32)
             + bhh)
    ig = jax.nn.sigmoid(gates[:, 0 * D:1 * D])
    fg = jax.nn.sigmoid(gates[:, 1 * D:2 * D])
    gg = jnp.tanh(gates[:, 2 * D:3 * D])
    og = jax.nn.sigmoid(gates[:, 3 * D:4 * D])
    cc2 = fg * cc + ig * gg
    return og * jnp.tanh(cc2), cc2


def _make_attn_body(with_mlp):
    def body(h_ref, b_ref, hh_ref, cc_ref, rp_ref, denp_ref,
             wih_ref, whh_ref, bih_ref, bhh_ref,
             w1_ref, b1_ref, w2_ref, b2_ref, w3_ref, b3_ref,
             r_ref, den_ref, hho_ref, cco_ref, out_ref, q_scr):
        i = pl.program_id(0)

        @pl.when(i == 0)
        def _():
            hh2, cc2 = _lstm_math(hh_ref[...], cc_ref[...], rp_ref[...],
                                  denp_ref[...], wih_ref[...], whh_ref[...],
                                  bih_ref[...], bhh_ref[...])
            hho_ref[...] = hh2
            cco_ref[...] = cc2
            q_scr[...] = hh2
            r_ref[...] = jnp.zeros_like(r_ref)
            den_ref[...] = jnp.zeros_like(den_ref)
        onehot = (b_ref[...] == lax.broadcasted_iota(jnp.int32, (_BN, B), 1)
                  ).astype(jnp.float32)
        h = h_ref[...]
        qb = jnp.dot(onehot, q_scr[...], precision=_PREC,
                     preferred_element_type=jnp.float32)
        e = jnp.sum(h * qb, axis=1, keepdims=True)
        a = jnp.exp(e)
        den_ref[...] += lax.dot_general(onehot, a, (((0,), (0,)), ((), ())),
                                        precision=_PREC,
                                        preferred_element_type=jnp.float32)
        r_ref[...] += lax.dot_general(onehot, a * h,
                                      (((0,), (0,)), ((), ())),
                                      precision=_PREC,
                                      preferred_element_type=jnp.float32)
        if with_mlp:
            @pl.when(i == _GRID - 1)
            def _():
                den = den_ref[...]
                r = r_ref[...] / jnp.where(den > 0.0, den, 1.0)
                q_star = jnp.concatenate([q_scr[...], r], axis=1)
                z = jnp.maximum(
                    jnp.dot(q_star, w1_ref[...], precision=_PREC,
                            preferred_element_type=jnp.float32)
                    + b1_ref[...], 0.0)
                z = jnp.maximum(
                    jnp.dot(z, w2_ref[...], precision=_PREC,
                            preferred_element_type=jnp.float32)
                    + b2_ref[...], 0.0)
                z = jnp.dot(z, w3_ref[...], precision=_PREC,
                            preferred_element_type=jnp.float32) + b3_ref[...]
                out_ref[...] = jax.nn.sigmoid(z)
        else:
            out_ref[...] = jnp.zeros_like(out_ref)
    return body


def _tc_attn_step(h, batch2, hh, cc, rp, denp, wih, whh, bih, bhh,
                  w1, b1, w2, b2, w3, b3, with_mlp):
    full = lambda i: (0, 0)
    return pl.pallas_call(
        _make_attn_body(with_mlp),
        grid=(_GRID,),
        in_specs=[
            pl.BlockSpec((_BN, D), lambda i: (i, 0)),
            pl.BlockSpec((_BN, 1), lambda i: (i, 0)),
            pl.BlockSpec((B, D), full),
            pl.BlockSpec((B, D), full),
            pl.BlockSpec((B, D), full),
            pl.BlockSpec((B, 1), full),
            pl.BlockSpec((4 * D, 2 * D), full),
            pl.BlockSpec((4 * D, D), full),
            pl.BlockSpec((1, 4 * D), full),
            pl.BlockSpec((1, 4 * D), full),
            pl.BlockSpec((2 * D, D), full),
            pl.BlockSpec((1, D), full),
            pl.BlockSpec((D, D // 2), full),
            pl.BlockSpec((1, D // 2), full),
            pl.BlockSpec((D // 2, 1), full),
            pl.BlockSpec((1, 1), full),
        ],
        out_specs=[
            pl.BlockSpec((B, D), full),
            pl.BlockSpec((B, 1), full),
            pl.BlockSpec((B, D), full),
            pl.BlockSpec((B, D), full),
            pl.BlockSpec((B, 1), full),
        ],
        out_shape=[
            jax.ShapeDtypeStruct((B, D), jnp.float32),
            jax.ShapeDtypeStruct((B, 1), jnp.float32),
            jax.ShapeDtypeStruct((B, D), jnp.float32),
            jax.ShapeDtypeStruct((B, D), jnp.float32),
            jax.ShapeDtypeStruct((B, 1), jnp.float32),
        ],
        scratch_shapes=[pltpu.VMEM((B, D), jnp.float32)],
    )(h, batch2, hh, cc, rp, denp, wih, whh, bih, bhh,
      w1, b1, w2, b2, w3, b3)


def _lstm_body(hh_ref, cc_ref, r_ref, den_ref, wih_ref, whh_ref,
               bih_ref, bhh_ref, hho_ref, cco_ref):
    hh = hh_ref[...]
    den = den_ref[...]
    r = r_ref[...] / jnp.where(den > 0.0, den, 1.0)
    q_star = jnp.concatenate([hh, r], axis=1)
    gates = (lax.dot_general(q_star, wih_ref[...], (((1,), (1,)), ((), ())),
                             precision=_PREC,
                             preferred_element_type=jnp.float32)
             + bih_ref[...]
             + lax.dot_general(hh, whh_ref[...], (((1,), (1,)), ((), ())),
                               precision=_PREC,
                               preferred_element_type=jnp.float32)
             + bhh_ref[...])
    ig = jax.nn.sigmoid(gates[:, 0 * D:1 * D])
    fg = jax.nn.sigmoid(gates[:, 1 * D:2 * D])
    gg = jnp.tanh(gates[:, 2 * D:3 * D])
    og = jax.nn.sigmoid(gates[:, 3 * D:4 * D])
    cc = fg * cc_ref[...] + ig * gg
    cco_ref[...] = cc
    hho_ref[...] = og * jnp.tanh(cc)


def _tc_lstm(hh, cc, r, den, wih, whh, bih, bhh):
    return pl.pallas_call(
        _lstm_body,
        out_shape=[
            jax.ShapeDtypeStruct((B, D), jnp.float32),
            jax.ShapeDtypeStruct((B, D), jnp.float32),
        ],
    )(hh, cc, r, den, wih, whh, bih, bhh)


def _mlp_body(hh_ref, r_ref, den_ref, w1_ref, b1_ref, w2_ref, b2_ref,
              w3_ref, b3_ref, out_ref):
    den = den_ref[...]
    r = r_ref[...] / jnp.where(den > 0.0, den, 1.0)
    q_star = jnp.concatenate([hh_ref[...], r], axis=1)
    z = jnp.maximum(jnp.dot(q_star, w1_ref[...], precision=_PREC,
                            preferred_element_type=jnp.float32)
                    + b1_ref[...], 0.0)
    z = jnp.maximum(jnp.dot(z, w2_ref[...], precision=_PREC,
                            preferred_element_type=jnp.float32)
                    + b2_ref[...], 0.0)
    z = jnp.dot(z, w3_ref[...], precision=_PREC,
                preferred_element_type=jnp.float32) + b3_ref[...]
    out_ref[...] = jax.nn.sigmoid(z)


def _tc_mlp(hh, r, den, w1, b1, w2, b2, w3, b3):
    return pl.pallas_call(
        _mlp_body,
        out_shape=jax.ShapeDtypeStruct((B, 1), jnp.float32),
    )(hh, r, den, w1, b1, w2, b2, w3, b3)


def kernel(x, edge_index, batch, emb, W_l0, b_l0, W_r0, W_l1, b_l1, W_r1,
           W_l2, b_l2, W_r2, W_ih, W_hh, b_ih, b_hh, Wc1, bc1, Wc2, bc2,
           Wc3, bc3):
    xpad = jnp.concatenate(
        [x[:, 0], jnp.zeros((PAD_N - N,), jnp.int32)])
    src = edge_index[0]
    dst = edge_index[1]
    batch2 = batch.reshape(N, 1)
    zeros8 = jnp.zeros((ZROWS, 8), jnp.float32)
    ones8 = jnp.ones((CE, 8), jnp.float32)
    zerosd = jnp.zeros((ZROWS, D), jnp.float32)

    h, degp = _sc_gather_deg(emb, xpad, dst, zeros8, ones8)

    for (Wl, bl, Wr) in ((W_l0, b_l0, W_r0), (W_l1, b_l1, W_r1),
                         (W_l2, b_l2, W_r2)):
        aggp = _sc_edge_agg(h, src, dst, zerosd)
        h = _tc_sage(aggp, degp, h, Wl, Wr, bl.reshape(1, D))

    hh = jnp.zeros((B, D), jnp.float32)
    cc = jnp.zeros((B, D), jnp.float32)
    r = jnp.zeros((B, D), jnp.float32)
    den = jnp.ones((B, 1), jnp.float32)
    bih2 = b_ih.reshape(1, 4 * D)
    bhh2 = b_hh.reshape(1, 4 * D)
    bc12 = bc1.reshape(1, D)
    bc22 = bc2.reshape(1, D // 2)
    bc32 = bc3.reshape(1, 1)
    out = None
    for step in range(STEPS):
        r, den, hh, cc, out = _tc_attn_step(
            h, batch2, hh, cc, r, den, W_ih, W_hh, bih2, bhh2,
            Wc1, bc12, Wc2, bc22, Wc3, bc32, step == STEPS - 1)
    return out[:, 0]
